# baseline (speedup 1.0000x reference)
import jax
import jax.numpy as jnp
from jax.experimental import pallas as pl
from jax.experimental.pallas import tpu as pltpu

B, L, N, H, NH = 4, 8192, 80, 768, 12
DH = H // NH
NPAD = 128
HPAD = 16
EPS = 1e-12
NEG = -1e30
BLK = 512


def _graph_kernel(g_ref, gT_ref, Wg_ref, WgT_ref, bg_ref, bgc_ref, AdT_ref,
                  hg_ref, edst_ref):
    g = g_ref[0]
    hg_ref[0] = jnp.dot(g, Wg_ref[...], preferred_element_type=jnp.float32) + bg_ref[...]
    hgT = jnp.dot(WgT_ref[...], gT_ref[0], preferred_element_type=jnp.float32) + bgc_ref[...]
    edT = jnp.dot(AdT_ref[...], hgT, preferred_element_type=jnp.float32)
    lane = jax.lax.broadcasted_iota(jnp.int32, (HPAD, NPAD), 1)
    edst_ref[0] = jnp.where(lane < N, edT, NEG)


def _attn_kernel(x_ref, s_ref, Wc_ref, bc_ref, As_ref, hg_ref, edst_ref,
                 ob_ref, Wo_ref, bo_ref, gam_ref, bet_ref, out_ref):
    x = x_ref[0]
    hc = jnp.dot(x, Wc_ref[...], preferred_element_type=jnp.float32) + bc_ref[...]
    e_src = jnp.dot(hc, As_ref[...], preferred_element_type=jnp.float32)
    sent = s_ref[0, 0]
    lane = jax.lax.broadcasted_iota(jnp.int32, (BLK, NPAD), 1)
    own = (sent == lane).astype(jnp.float32)
    ctx_parts = []
    for h in range(NH):
        s = e_src[:, h:h + 1] + edst_ref[0, h:h + 1, :]
        s = jnp.where(s >= 0, s, 0.2 * s)
        s = s + own * ob_ref[0:1, h:h + 1]
        m = jnp.max(s, axis=1, keepdims=True)
        e = jnp.exp(s - m)
        denom = jnp.sum(e, axis=1, keepdims=True)
        hg_h = hg_ref[0, :, h * DH:(h + 1) * DH]
        ctx_parts.append(
            jnp.dot(e, hg_h, preferred_element_type=jnp.float32) / denom)
    ctx = jnp.concatenate(ctx_parts, axis=1)
    upd = jnp.dot(ctx, Wo_ref[...], preferred_element_type=jnp.float32) + bo_ref[...]
    xr = x + upd
    mu = jnp.mean(xr, axis=1, keepdims=True)
    xc = xr - mu
    var = jnp.mean(xc * xc, axis=1, keepdims=True)
    out_ref[0] = gam_ref[...] * xc * jax.lax.rsqrt(var + EPS) + bet_ref[...]


@jax.jit
def kernel(context_vectors, graph_vectors, sent_ind, Wc, bc, Wg, bg,
           a_src, a_dst, own_bias, Wo, bo, ln_gamma, ln_beta):
    nblk = L // BLK
    eye = jnp.eye(NH, dtype=jnp.float32)
    A_src = (a_src[:, :, None] * eye[:, None, :]).reshape(H, NH)
    A_src = jnp.pad(A_src, ((0, 0), (0, NPAD - NH)))
    AdT = (eye[:, :, None] * a_dst[None, :, :]).reshape(NH, H)
    AdT = jnp.pad(AdT, ((0, HPAD - NH), (0, 0)))
    ob = jnp.pad(own_bias, (0, HPAD - NH)).reshape(1, HPAD)
    g_pad = jnp.pad(graph_vectors, ((0, 0), (0, NPAD - N), (0, 0)))
    gT_pad = jnp.transpose(g_pad, (0, 2, 1))
    bg_col = jnp.broadcast_to(bg[:, None], (H, NPAD))
    row = lambda v: v.reshape(1, H)
    sent4 = sent_ind.reshape(B, nblk, BLK, 1)

    hg, edst = pl.pallas_call(
        _graph_kernel,
        grid=(B,),
        in_specs=[
            pl.BlockSpec((1, NPAD, H), lambda b: (b, 0, 0)),
            pl.BlockSpec((1, H, NPAD), lambda b: (b, 0, 0)),
            pl.BlockSpec((H, H), lambda b: (0, 0)),
            pl.BlockSpec((H, H), lambda b: (0, 0)),
            pl.BlockSpec((1, H), lambda b: (0, 0)),
            pl.BlockSpec((H, NPAD), lambda b: (0, 0)),
            pl.BlockSpec((HPAD, H), lambda b: (0, 0)),
        ],
        out_specs=[
            pl.BlockSpec((1, NPAD, H), lambda b: (b, 0, 0)),
            pl.BlockSpec((1, HPAD, NPAD), lambda b: (b, 0, 0)),
        ],
        out_shape=[
            jax.ShapeDtypeStruct((B, NPAD, H), jnp.float32),
            jax.ShapeDtypeStruct((B, HPAD, NPAD), jnp.float32),
        ],
    )(g_pad, gT_pad, Wg, Wg.T, row(bg), bg_col, AdT)

    out = pl.pallas_call(
        _attn_kernel,
        grid=(B, nblk),
        in_specs=[
            pl.BlockSpec((1, BLK, H), lambda b, i: (b, i, 0)),
            pl.BlockSpec((1, 1, BLK, 1), lambda b, i: (b, i, 0, 0)),
            pl.BlockSpec((H, H), lambda b, i: (0, 0)),
            pl.BlockSpec((1, H), lambda b, i: (0, 0)),
            pl.BlockSpec((H, NPAD), lambda b, i: (0, 0)),
            pl.BlockSpec((1, NPAD, H), lambda b, i: (b, 0, 0)),
            pl.BlockSpec((1, HPAD, NPAD), lambda b, i: (b, 0, 0)),
            pl.BlockSpec((1, HPAD), lambda b, i: (0, 0)),
            pl.BlockSpec((H, H), lambda b, i: (0, 0)),
            pl.BlockSpec((1, H), lambda b, i: (0, 0)),
            pl.BlockSpec((1, H), lambda b, i: (0, 0)),
            pl.BlockSpec((1, H), lambda b, i: (0, 0)),
        ],
        out_specs=pl.BlockSpec((1, BLK, H), lambda b, i: (b, i, 0)),
        out_shape=jax.ShapeDtypeStruct((B, L, H), jnp.float32),
    )(context_vectors,
      sent4, Wc, row(bc), A_src, hg, edst, ob, Wo, row(bo),
      row(ln_gamma), row(ln_beta))
    return out


# speedup vs baseline: 1.9974x; 1.9974x over previous
import jax
import jax.numpy as jnp
from jax.experimental import pallas as pl
from jax.experimental.pallas import tpu as pltpu

B, L, N, H, NH = 4, 8192, 80, 768, 12
DH = H // NH
NPAD = 128
PACK = NH * NPAD
HPAD = 16
EPS = 1e-12
NEG = -1e30
BLK = 512


def _graph_kernel(g_ref, gT_ref, Wg_ref, WgT_ref, bg_ref, bgc_ref, AdT_ref,
                  Wo_ref, ed_ref, W2_ref):
    g = g_ref[0]
    hg = jnp.dot(g, Wg_ref[...], preferred_element_type=jnp.float32) + bg_ref[...]
    hgT = jnp.dot(WgT_ref[...], gT_ref[0], preferred_element_type=jnp.float32) + bgc_ref[...]
    lane = jax.lax.broadcasted_iota(jnp.int32, (1, NPAD), 1)
    for h in range(NH):
        ed_h = jnp.dot(AdT_ref[h:h + 1, :], hgT, preferred_element_type=jnp.float32)
        ed_ref[0, 0:1, h * NPAD:(h + 1) * NPAD] = jnp.where(lane < N, ed_h, NEG)
        w2_h = jnp.dot(hg[:, h * DH:(h + 1) * DH].astype(jnp.bfloat16),
                       Wo_ref[h * DH:(h + 1) * DH, :].astype(jnp.bfloat16),
                       preferred_element_type=jnp.float32)
        W2_ref[0, h * NPAD:(h + 1) * NPAD, :] = w2_h.astype(jnp.bfloat16)


def _attn_kernel(x_ref, s_ref, ws_ref, pexp_ref, segT_ref, noc_ref, boc_ref,
                 ed_ref, W2_ref, bo_ref, gam_ref, bet_ref, out_ref):
    x = x_ref[0]
    es = jnp.dot(x.astype(jnp.bfloat16), ws_ref[...],
                 preferred_element_type=jnp.float32)
    s = jnp.dot(es.astype(jnp.bfloat16), pexp_ref[...],
                preferred_element_type=jnp.float32)
    s = s + ed_ref[0]
    s = jnp.maximum(s, 0.2 * s)
    sent = s_ref[0, 0]
    s = s + jnp.where(sent == noc_ref[...], boc_ref[...], 0.0)
    m = jnp.max(s, axis=1, keepdims=True)
    e = jnp.exp(s - m)
    denom = jnp.dot(e.astype(jnp.bfloat16), segT_ref[...],
                    preferred_element_type=jnp.float32)
    rb = jnp.dot((1.0 / jnp.maximum(denom, 1e-30)).astype(jnp.bfloat16), pexp_ref[...],
                 preferred_element_type=jnp.float32)
    alpha = (e * rb).astype(jnp.bfloat16)
    upd = jnp.dot(alpha, W2_ref[0], preferred_element_type=jnp.float32) + bo_ref[...]
    xr = x + upd
    mu = jnp.mean(xr, axis=1, keepdims=True)
    xc = xr - mu
    var = jnp.mean(xc * xc, axis=1, keepdims=True)
    out_ref[0] = gam_ref[...] * xc * jax.lax.rsqrt(var + EPS) + bet_ref[...]


@jax.jit
def kernel(context_vectors, graph_vectors, sent_ind, Wc, bc, Wg, bg,
           a_src, a_dst, own_bias, Wo, bo, ln_gamma, ln_beta):
    nblk = L // BLK
    f32, bf16 = jnp.float32, jnp.bfloat16
    eye = jnp.eye(NH, dtype=f32)
    A_src = (a_src[:, :, None] * eye[:, None, :]).reshape(H, NH)
    ws = jnp.pad((Wc @ A_src), ((0, 0), (0, NPAD - NH))).astype(bf16)
    es_bias = bc @ A_src
    AdT = jnp.pad((eye[:, :, None] * a_dst[None, :, :]).reshape(NH, H),
                  ((0, HPAD - NH), (0, 0)))
    hsel = (jnp.arange(NPAD)[:, None] == jnp.arange(NH)[None, :]).astype(f32)
    pexp = jnp.broadcast_to(hsel[:, :, None], (NPAD, NH, NPAD))
    pexp = pexp.reshape(NPAD, PACK).astype(bf16)
    segT = (jnp.arange(NH)[:, None] == jnp.arange(NPAD)[None, :]).astype(f32)
    segT = jnp.broadcast_to(segT[:, None, :], (NH, NPAD, NPAD))
    segT = segT.reshape(PACK, NPAD).astype(bf16)
    noc = jnp.tile(jnp.arange(NPAD, dtype=jnp.int32), NH).reshape(1, PACK)
    boc = jnp.repeat(own_bias, NPAD).reshape(1, PACK).astype(f32)
    g_pad = jnp.pad(graph_vectors, ((0, 0), (0, NPAD - N), (0, 0)))
    gT_pad = jnp.transpose(g_pad, (0, 2, 1))
    bg_col = jnp.broadcast_to(bg[:, None], (H, NPAD))
    row = lambda v: v.reshape(1, H)
    sent4 = sent_ind.reshape(B, nblk, BLK, 1)

    ed, W2 = pl.pallas_call(
        _graph_kernel,
        grid=(B,),
        in_specs=[
            pl.BlockSpec((1, NPAD, H), lambda b: (b, 0, 0)),
            pl.BlockSpec((1, H, NPAD), lambda b: (b, 0, 0)),
            pl.BlockSpec((H, H), lambda b: (0, 0)),
            pl.BlockSpec((H, H), lambda b: (0, 0)),
            pl.BlockSpec((1, H), lambda b: (0, 0)),
            pl.BlockSpec((H, NPAD), lambda b: (0, 0)),
            pl.BlockSpec((HPAD, H), lambda b: (0, 0)),
            pl.BlockSpec((H, H), lambda b: (0, 0)),
        ],
        out_specs=[
            pl.BlockSpec((1, 1, PACK), lambda b: (b, 0, 0)),
            pl.BlockSpec((1, PACK, H), lambda b: (b, 0, 0)),
        ],
        out_shape=[
            jax.ShapeDtypeStruct((B, 1, PACK), f32),
            jax.ShapeDtypeStruct((B, PACK, H), bf16),
        ],
    )(g_pad, gT_pad, Wg, Wg.T, row(bg), bg_col, AdT, Wo)

    ed = ed + jnp.repeat(es_bias.astype(f32), NPAD).reshape(1, 1, PACK)

    out = pl.pallas_call(
        _attn_kernel,
        grid=(B, nblk),
        in_specs=[
            pl.BlockSpec((1, BLK, H), lambda b, i: (b, i, 0)),
            pl.BlockSpec((1, 1, BLK, 1), lambda b, i: (b, i, 0, 0)),
            pl.BlockSpec((H, NPAD), lambda b, i: (0, 0)),
            pl.BlockSpec((NPAD, PACK), lambda b, i: (0, 0)),
            pl.BlockSpec((PACK, NPAD), lambda b, i: (0, 0)),
            pl.BlockSpec((1, PACK), lambda b, i: (0, 0)),
            pl.BlockSpec((1, PACK), lambda b, i: (0, 0)),
            pl.BlockSpec((1, 1, PACK), lambda b, i: (b, 0, 0)),
            pl.BlockSpec((1, PACK, H), lambda b, i: (b, 0, 0)),
            pl.BlockSpec((1, H), lambda b, i: (0, 0)),
            pl.BlockSpec((1, H), lambda b, i: (0, 0)),
            pl.BlockSpec((1, H), lambda b, i: (0, 0)),
        ],
        out_specs=pl.BlockSpec((1, BLK, H), lambda b, i: (b, i, 0)),
        out_shape=jax.ShapeDtypeStruct((B, L, H), f32),
    )(context_vectors, sent4, ws, pexp, segT, noc, boc, ed, W2,
      row(bo), row(ln_gamma), row(ln_beta))
    return out


# speedup vs baseline: 2.6022x; 1.3028x over previous
import jax
import jax.numpy as jnp
from jax.experimental import pallas as pl
from jax.experimental.pallas import tpu as pltpu

B, L, N, H, NH = 4, 8192, 80, 768, 12
DH = H // NH
NPAD = 128
NP = 96
PACK = NH * NP
HPAD = 16
EPS = 1e-12
NEG = -1e30
BLK = 1024


def _graph_kernel(g_ref, gT_ref, Wg_ref, WgT_ref, bg_ref, bgc_ref, AdT_ref,
                  Wo_ref, Sh_ref, ed_ref, W2_ref):
    g = g_ref[0]
    hg = jnp.dot(g, Wg_ref[...], preferred_element_type=jnp.float32) + bg_ref[...]
    hgT = jnp.dot(WgT_ref[...], gT_ref[0], preferred_element_type=jnp.float32) + bgc_ref[...]
    lane = jax.lax.broadcasted_iota(jnp.int32, (1, NPAD), 1)
    acc = jnp.zeros((1, PACK), jnp.float32)
    for h in range(NH):
        ed_h = jnp.dot(AdT_ref[h:h + 1, :], hgT, preferred_element_type=jnp.float32)
        ed_h = jnp.where(lane < N, ed_h, NEG)
        acc = acc + jnp.dot(ed_h, Sh_ref[h], preferred_element_type=jnp.float32)
        w2_h = jnp.dot(hg[:NP, h * DH:(h + 1) * DH].astype(jnp.bfloat16),
                       Wo_ref[h * DH:(h + 1) * DH, :].astype(jnp.bfloat16),
                       preferred_element_type=jnp.float32)
        W2_ref[0, h * NP:(h + 1) * NP, :] = w2_h.astype(jnp.bfloat16)
    ed_ref[0] = acc


def _attn_kernel(x_ref, s_ref, ws_ref, pexp_ref, segT_ref, noc_ref, boc_ref,
                 ed_ref, W2_ref, bo_ref, gam_ref, bet_ref, out_ref):
    x = x_ref[0]
    es = jnp.dot(x.astype(jnp.bfloat16), ws_ref[...],
                 preferred_element_type=jnp.float32)
    m = jnp.max(es, axis=1, keepdims=True)
    s = jnp.dot(es.astype(jnp.bfloat16), pexp_ref[...],
                preferred_element_type=jnp.float32)
    s = s + ed_ref[0]
    s = jnp.maximum(s, 0.2 * s)
    sent = s_ref[0, 0]
    s = s + jnp.where(sent == noc_ref[...], boc_ref[...], 0.0)
    e = jnp.exp(s - m)
    denom = jnp.dot(e.astype(jnp.bfloat16), segT_ref[...],
                    preferred_element_type=jnp.float32)
    rb = jnp.dot((1.0 / jnp.maximum(denom, 1e-30)).astype(jnp.bfloat16),
                 pexp_ref[...], preferred_element_type=jnp.float32)
    alpha = (e * rb).astype(jnp.bfloat16)
    upd = jnp.dot(alpha, W2_ref[0], preferred_element_type=jnp.float32) + bo_ref[...]
    xr = x + upd
    mu = jnp.mean(xr, axis=1, keepdims=True)
    xc = xr - mu
    var = jnp.mean(xc * xc, axis=1, keepdims=True)
    out_ref[0] = gam_ref[...] * xc * jax.lax.rsqrt(var + EPS) + bet_ref[...]


@jax.jit
def kernel(context_vectors, graph_vectors, sent_ind, Wc, bc, Wg, bg,
           a_src, a_dst, own_bias, Wo, bo, ln_gamma, ln_beta):
    nblk = L // BLK
    f32, bf16 = jnp.float32, jnp.bfloat16
    eye = jnp.eye(NH, dtype=f32)
    A_src = (a_src[:, :, None] * eye[:, None, :]).reshape(H, NH)
    ws = jnp.pad((Wc @ A_src), ((0, 0), (0, NPAD - NH))).astype(bf16)
    es_bias = bc @ A_src
    AdT = jnp.pad((eye[:, :, None] * a_dst[None, :, :]).reshape(NH, H),
                  ((0, HPAD - NH), (0, 0)))
    h_of_c = jnp.repeat(jnp.arange(NH), NP)
    n_of_c = jnp.tile(jnp.arange(NP), NH)
    pexp = (jnp.arange(NPAD)[:, None] == h_of_c[None, :]).astype(bf16)
    segT = (h_of_c[:, None] == jnp.arange(NPAD)[None, :]).astype(bf16)
    Sh = ((jnp.arange(NPAD)[None, :, None] == n_of_c[None, None, :])
          & (h_of_c[None, None, :] == jnp.arange(NH)[:, None, None])
          ).astype(f32)
    noc = n_of_c.astype(jnp.int32).reshape(1, PACK)
    boc = jnp.repeat(own_bias, NP).reshape(1, PACK).astype(f32)
    g_pad = jnp.pad(graph_vectors, ((0, 0), (0, NPAD - N), (0, 0)))
    gT_pad = jnp.transpose(g_pad, (0, 2, 1))
    bg_col = jnp.broadcast_to(bg[:, None], (H, NPAD))
    row = lambda v: v.reshape(1, H)
    sent4 = sent_ind.reshape(B, nblk, BLK, 1)

    ed, W2 = pl.pallas_call(
        _graph_kernel,
        grid=(B,),
        in_specs=[
            pl.BlockSpec((1, NPAD, H), lambda b: (b, 0, 0)),
            pl.BlockSpec((1, H, NPAD), lambda b: (b, 0, 0)),
            pl.BlockSpec((H, H), lambda b: (0, 0)),
            pl.BlockSpec((H, H), lambda b: (0, 0)),
            pl.BlockSpec((1, H), lambda b: (0, 0)),
            pl.BlockSpec((H, NPAD), lambda b: (0, 0)),
            pl.BlockSpec((HPAD, H), lambda b: (0, 0)),
            pl.BlockSpec((H, H), lambda b: (0, 0)),
            pl.BlockSpec((NH, NPAD, PACK), lambda b: (0, 0, 0)),
        ],
        out_specs=[
            pl.BlockSpec((1, 1, PACK), lambda b: (b, 0, 0)),
            pl.BlockSpec((1, PACK, H), lambda b: (b, 0, 0)),
        ],
        out_shape=[
            jax.ShapeDtypeStruct((B, 1, PACK), f32),
            jax.ShapeDtypeStruct((B, PACK, H), bf16),
        ],
    )(g_pad, gT_pad, Wg, Wg.T, row(bg), bg_col, AdT, Wo, Sh)

    ed = ed + jnp.repeat(es_bias.astype(f32), NP).reshape(1, 1, PACK)

    out = pl.pallas_call(
        _attn_kernel,
        grid=(B, nblk),
        in_specs=[
            pl.BlockSpec((1, BLK, H), lambda b, i: (b, i, 0)),
            pl.BlockSpec((1, 1, BLK, 1), lambda b, i: (b, i, 0, 0)),
            pl.BlockSpec((H, NPAD), lambda b, i: (0, 0)),
            pl.BlockSpec((NPAD, PACK), lambda b, i: (0, 0)),
            pl.BlockSpec((PACK, NPAD), lambda b, i: (0, 0)),
            pl.BlockSpec((1, PACK), lambda b, i: (0, 0)),
            pl.BlockSpec((1, PACK), lambda b, i: (0, 0)),
            pl.BlockSpec((1, 1, PACK), lambda b, i: (b, 0, 0)),
            pl.BlockSpec((1, PACK, H), lambda b, i: (b, 0, 0)),
            pl.BlockSpec((1, H), lambda b, i: (0, 0)),
            pl.BlockSpec((1, H), lambda b, i: (0, 0)),
            pl.BlockSpec((1, H), lambda b, i: (0, 0)),
        ],
        out_specs=pl.BlockSpec((1, BLK, H), lambda b, i: (b, i, 0)),
        out_shape=jax.ShapeDtypeStruct((B, L, H), f32),
    )(context_vectors, sent4, ws, pexp, segT, noc, boc, ed, W2,
      row(bo), row(ln_gamma), row(ln_beta))
    return out


# speedup vs baseline: 2.9547x; 1.1355x over previous
import jax
import jax.numpy as jnp
from jax.experimental import pallas as pl
from jax.experimental.pallas import tpu as pltpu

B, L, N, H, NH = 4, 8192, 80, 768, 12
DH = H // NH
NPAD = 128
PACK = NH * N
HPAD = 16
AUG = 12
EPS = 1e-12
NEG = -1e30
BLK = 1024


def _graph_kernel(g_ref, gT_ref, Wg_ref, WgT_ref, bg_ref, bgc_ref, AdT_ref,
                  Wo_ref, Sh_ref, ed_ref, W2_ref):
    g = g_ref[0]
    hg = jnp.dot(g, Wg_ref[...], preferred_element_type=jnp.float32) + bg_ref[...]
    hgT = jnp.dot(WgT_ref[...], gT_ref[0], preferred_element_type=jnp.float32) + bgc_ref[...]
    acc = jnp.zeros((1, PACK), jnp.float32)
    for h in range(NH):
        ed_h = jnp.dot(AdT_ref[h:h + 1, :], hgT, preferred_element_type=jnp.float32)
        acc = acc + jnp.dot(ed_h, Sh_ref[h], preferred_element_type=jnp.float32)
        w2_h = jnp.dot(hg[:N, h * DH:(h + 1) * DH].astype(jnp.bfloat16),
                       Wo_ref[h * DH:(h + 1) * DH, :].astype(jnp.bfloat16),
                       preferred_element_type=jnp.float32)
        W2_ref[0, h * N:(h + 1) * N, :] = w2_h.astype(jnp.bfloat16)
    ed_ref[0] = acc


def _attn_kernel(x_ref, s_ref, ws_ref, c12_ref, pexpa_ref, pexp_ref, segT_ref,
                 noc_ref, boc_ref, W2_ref, bo_ref, gam_ref, bet_ref, out_ref):
    bf16 = jnp.bfloat16
    x = x_ref[0]
    es = jnp.dot(x.astype(bf16), ws_ref[...],
                 preferred_element_type=jnp.float32) + c12_ref[...]
    m = jnp.max(es, axis=1, keepdims=True)
    s = jnp.dot(es.astype(bf16), pexpa_ref[0],
                preferred_element_type=jnp.float32).astype(bf16)
    s = jnp.maximum(s, bf16(0.2) * s)
    sent = s_ref[0, 0]
    s = s + jnp.where(sent == noc_ref[...], boc_ref[...], bf16(0.0))
    e = jnp.exp(s - m.astype(bf16))
    denom = jnp.dot(e, segT_ref[...],
                    preferred_element_type=jnp.float32)
    rb = jnp.dot((1.0 / jnp.maximum(denom, 1e-30)).astype(bf16), pexp_ref[...],
                 preferred_element_type=jnp.float32).astype(bf16)
    alpha = e * rb
    upd = jnp.dot(alpha, W2_ref[0], preferred_element_type=jnp.float32) + bo_ref[...]
    xr = x + upd
    mu = jnp.mean(xr, axis=1, keepdims=True)
    xc = xr - mu
    var = jnp.mean(xc * xc, axis=1, keepdims=True)
    out_ref[0] = gam_ref[...] * xc * jax.lax.rsqrt(var + EPS) + bet_ref[...]


@jax.jit
def kernel(context_vectors, graph_vectors, sent_ind, Wc, bc, Wg, bg,
           a_src, a_dst, own_bias, Wo, bo, ln_gamma, ln_beta):
    nblk = L // BLK
    f32, bf16 = jnp.float32, jnp.bfloat16
    eye = jnp.eye(NH, dtype=f32)
    A_src = (a_src[:, :, None] * eye[:, None, :]).reshape(H, NH)
    ws = jnp.pad((Wc @ A_src), ((0, 0), (0, NPAD - NH))).astype(bf16)
    es_bias = bc @ A_src
    AdT = jnp.pad((eye[:, :, None] * a_dst[None, :, :]).reshape(NH, H),
                  ((0, HPAD - NH), (0, 0)))
    h_of_c = jnp.repeat(jnp.arange(NH), N)
    n_of_c = jnp.tile(jnp.arange(N), NH)
    pexp = (jnp.arange(NPAD)[:, None] == h_of_c[None, :]).astype(bf16)
    segT = (h_of_c[:, None] == jnp.arange(NPAD)[None, :]).astype(bf16)
    Sh = ((jnp.arange(NPAD)[None, :, None] == n_of_c[None, None, :])
          & (h_of_c[None, None, :] == jnp.arange(NH)[:, None, None])
          ).astype(f32)
    c12 = (jnp.arange(NPAD) == AUG).astype(f32).reshape(1, NPAD)
    noc = n_of_c.astype(bf16).reshape(1, PACK)
    boc = jnp.repeat(own_bias, N).reshape(1, PACK).astype(bf16)
    g_pad = jnp.pad(graph_vectors, ((0, 0), (0, NPAD - N), (0, 0)))
    gT_pad = jnp.transpose(g_pad, (0, 2, 1))
    bg_col = jnp.broadcast_to(bg[:, None], (H, NPAD))
    row = lambda v: v.reshape(1, H)
    sent4 = sent_ind.astype(bf16).reshape(B, nblk, BLK, 1)

    ed, W2 = pl.pallas_call(
        _graph_kernel,
        grid=(B,),
        in_specs=[
            pl.BlockSpec((1, NPAD, H), lambda b: (b, 0, 0)),
            pl.BlockSpec((1, H, NPAD), lambda b: (b, 0, 0)),
            pl.BlockSpec((H, H), lambda b: (0, 0)),
            pl.BlockSpec((H, H), lambda b: (0, 0)),
            pl.BlockSpec((1, H), lambda b: (0, 0)),
            pl.BlockSpec((H, NPAD), lambda b: (0, 0)),
            pl.BlockSpec((HPAD, H), lambda b: (0, 0)),
            pl.BlockSpec((H, H), lambda b: (0, 0)),
            pl.BlockSpec((NH, NPAD, PACK), lambda b: (0, 0, 0)),
        ],
        out_specs=[
            pl.BlockSpec((1, 1, PACK), lambda b: (b, 0, 0)),
            pl.BlockSpec((1, PACK, H), lambda b: (b, 0, 0)),
        ],
        out_shape=[
            jax.ShapeDtypeStruct((B, 1, PACK), f32),
            jax.ShapeDtypeStruct((B, PACK, H), bf16),
        ],
    )(g_pad, gT_pad, Wg, Wg.T, row(bg), bg_col, AdT, Wo, Sh)

    ed_full = ed[:, 0, :] + jnp.repeat(es_bias.astype(f32), N)[None, :]
    pexp_aug = jnp.broadcast_to(pexp[None].astype(f32), (B, NPAD, PACK))
    pexp_aug = pexp_aug.at[:, AUG, :].set(ed_full).astype(bf16)

    out = pl.pallas_call(
        _attn_kernel,
        grid=(B, nblk),
        in_specs=[
            pl.BlockSpec((1, BLK, H), lambda b, i: (b, i, 0)),
            pl.BlockSpec((1, 1, BLK, 1), lambda b, i: (b, i, 0, 0)),
            pl.BlockSpec((H, NPAD), lambda b, i: (0, 0)),
            pl.BlockSpec((1, NPAD), lambda b, i: (0, 0)),
            pl.BlockSpec((1, NPAD, PACK), lambda b, i: (b, 0, 0)),
            pl.BlockSpec((NPAD, PACK), lambda b, i: (0, 0)),
            pl.BlockSpec((PACK, NPAD), lambda b, i: (0, 0)),
            pl.BlockSpec((1, PACK), lambda b, i: (0, 0)),
            pl.BlockSpec((1, PACK), lambda b, i: (0, 0)),
            pl.BlockSpec((1, PACK, H), lambda b, i: (b, 0, 0)),
            pl.BlockSpec((1, H), lambda b, i: (0, 0)),
            pl.BlockSpec((1, H), lambda b, i: (0, 0)),
            pl.BlockSpec((1, H), lambda b, i: (0, 0)),
        ],
        out_specs=pl.BlockSpec((1, BLK, H), lambda b, i: (b, i, 0)),
        out_shape=jax.ShapeDtypeStruct((B, L, H), f32),
    )(context_vectors, sent4, ws, c12, pexp_aug, pexp, segT, noc, boc, W2,
      row(bo), row(ln_gamma), row(ln_beta))
    return out
